# baseline (speedup 1.0000x reference)
import functools

import jax
import jax.numpy as jnp
from jax import lax
from jax.experimental import pallas as pl
from jax.experimental.pallas import tpu as pltpu
from jax.experimental.pallas import tpu_sc as plsc

N = 10000
NP = 10240
E = 320000
HID = 128
T = 100000
TP = 102400
OUT = 2
CHK = 80

NC = 2
NS = 16
NW = NC * NS

EPW = E // NW
ECH = EPW // CHK
HRW = NP // NW
HCH = HRW // CHK
TPW = TP // NW
TCH = TPW // CHK
SRS = NP // NS
SCH = SRS // CHK


def _zero_rows(ref, nrows, width):
    zv = jnp.zeros((16,), jnp.float32)

    def body(r, carry):
        for f in range(width // 16):
            ref[r, pl.ds(f * 16, 16)] = zv
        return carry

    lax.fori_loop(0, nrows, body, 0)


@functools.cache
def _sc_kernels():
    mesh = plsc.VectorSubcoreMesh(core_axis_name="c", subcore_axis_name="s",
                                  num_cores=NC, num_subcores=NS)

    @functools.partial(
        pl.kernel,
        out_type=jax.ShapeDtypeStruct((NP, HID), jnp.float32),
        mesh=mesh,
        scratch_types=[
            pltpu.VMEM((CHK,), jnp.int32),
            pltpu.VMEM((CHK, HID), jnp.float32),
            pltpu.SemaphoreType.DMA,
        ],
    )
    def prep(nf_hbm, emb_hbm, h_hbm, idx_v, rows_v, sem):
        c = lax.axis_index("c")
        s = lax.axis_index("s")
        wid = c * NS + s

        def hbody(k, carry):
            base = (wid * HCH + k) * CHK
            pltpu.sync_copy(nf_hbm.at[pl.ds(base, CHK)], idx_v)
            pltpu.async_copy(emb_hbm.at[idx_v], rows_v, sem).wait()
            pltpu.sync_copy(rows_v, h_hbm.at[pl.ds(base, CHK)])
            return carry

        lax.fori_loop(0, HCH, hbody, 0)

    def make_segsum(num_edges):
      epw = num_edges // NW
      ech = epw // CHK

      @functools.partial(
        pl.kernel,
        out_type=jax.ShapeDtypeStruct((NC, NP, HID), jnp.float32),
        mesh=mesh,
        scratch_types=[
            pltpu.VMEM((CHK,), jnp.int32),
            pltpu.VMEM((CHK,), jnp.int32),
            pltpu.VMEM((CHK, HID), jnp.float32),
            pltpu.VMEM_SHARED((NP, HID), jnp.float32),
            pltpu.SemaphoreType.DMA,
        ],
      )
      def segsum(src_hbm, dst_hbm, hn_hbm, part_hbm, si_v, di_v, rows_v,
                 acc_sh, sem):
        c = lax.axis_index("c")
        s = lax.axis_index("s")
        wid = c * NS + s
        base_r = s * SRS

        _zero_rows(rows_v, CHK, HID)

        def zbody(j, carry):
            pltpu.sync_copy(rows_v, acc_sh.at[pl.ds(base_r + j * CHK, CHK)])
            return carry

        lax.fori_loop(0, SCH, zbody, 0)
        plsc.subcore_barrier()

        def ebody(i, carry):
            e0 = (wid * ech + i) * CHK
            pltpu.sync_copy(src_hbm.at[pl.ds(e0, CHK)], si_v)
            pltpu.sync_copy(dst_hbm.at[pl.ds(e0, CHK)], di_v)
            pltpu.async_copy(hn_hbm.at[si_v], rows_v, sem).wait()
            pltpu.sync_copy(rows_v, acc_sh.at[di_v], add=True)
            return carry

        lax.fori_loop(0, ech, ebody, 0)
        plsc.subcore_barrier()

        def obody(j, carry):
            r0 = base_r + j * CHK
            pltpu.sync_copy(acc_sh.at[pl.ds(r0, CHK)], rows_v)
            pltpu.sync_copy(rows_v, part_hbm.at[c, pl.ds(r0, CHK)])
            return carry

        lax.fori_loop(0, SCH, obody, 0)

      return segsum

    segsum = make_segsum(E)
    cntsum = make_segsum(2 * E)

    @functools.partial(
        pl.kernel,
        out_type=jax.ShapeDtypeStruct((TP, HID), jnp.float32),
        mesh=mesh,
        scratch_types=[
            pltpu.VMEM((CHK,), jnp.int32),
            pltpu.VMEM((CHK,), jnp.int32),
            pltpu.VMEM((CHK, HID), jnp.float32),
            pltpu.VMEM((CHK, HID), jnp.float32),
            pltpu.SemaphoreType.DMA,
        ],
    )
    def mlp_gather(ts_hbm, td_hbm, p1_hbm, p2_hbm, z_hbm, i1_v, i2_v,
                   g1_v, g2_v, sem):
        c = lax.axis_index("c")
        s = lax.axis_index("s")
        wid = c * NS + s

        def tbody(i, carry):
            t0 = (wid * TCH + i) * CHK
            pltpu.sync_copy(ts_hbm.at[pl.ds(t0, CHK)], i1_v)
            pltpu.sync_copy(td_hbm.at[pl.ds(t0, CHK)], i2_v)
            pltpu.async_copy(p1_hbm.at[i1_v], g1_v, sem).wait()
            pltpu.async_copy(p2_hbm.at[i2_v], g2_v, sem).wait()

            def rbody(r, carry2):
                for f in range(HID // 16):
                    sl = pl.ds(f * 16, 16)
                    g1_v[r, sl] = jnp.maximum(g1_v[r, sl] + g2_v[r, sl], 0.0)
                return carry2

            lax.fori_loop(0, CHK, rbody, 0)
            pltpu.sync_copy(g1_v, z_hbm.at[pl.ds(t0, CHK)])
            return carry

        lax.fori_loop(0, TCH, tbody, 0)

    return prep, segsum, cntsum, mlp_gather


def _norms_body(cnt_ref, h_ref, hn_ref, ns_ref, nd_ref):
    degs = cnt_ref[0, :, 0:1]
    degd = cnt_ref[0, :, 1:2]
    for cc in range(1, NC):
        degs = degs + cnt_ref[cc, :, 0:1]
        degd = degd + cnt_ref[cc, :, 1:2]
    ns = lax.rsqrt(jnp.maximum(degs, 1.0))
    nd = lax.rsqrt(jnp.maximum(degd, 1.0))
    ns_ref[...] = ns
    nd_ref[...] = nd
    hn_ref[...] = h_ref[...] * ns


def _norms_tc(cnts, h):
    return pl.pallas_call(
        _norms_body,
        out_shape=[
            jax.ShapeDtypeStruct((NP, HID), jnp.float32),
            jax.ShapeDtypeStruct((NP, 1), jnp.float32),
            jax.ShapeDtypeStruct((NP, 1), jnp.float32),
        ],
    )(cnts, h)


_RB = 1280


def _upd0_body(p_ref, nd_ref, ns_ref, w_ref, b_ref, o_ref):
    agg = p_ref[0]
    for cc in range(1, NC):
        agg = agg + p_ref[cc]
    agg = agg * nd_ref[...]
    h = jnp.dot(agg, w_ref[...], preferred_element_type=jnp.float32) + b_ref[...]
    o_ref[...] = h * ns_ref[...]


def _upd0(part, nd, ns, w, b):
    grid = (NP // _RB,)
    return pl.pallas_call(
        _upd0_body,
        grid=grid,
        in_specs=[
            pl.BlockSpec((NC, _RB, HID), lambda i: (0, i, 0)),
            pl.BlockSpec((_RB, 1), lambda i: (i, 0)),
            pl.BlockSpec((_RB, 1), lambda i: (i, 0)),
            pl.BlockSpec((HID, HID), lambda i: (0, 0)),
            pl.BlockSpec((1, HID), lambda i: (0, 0)),
        ],
        out_specs=pl.BlockSpec((_RB, HID), lambda i: (i, 0)),
        out_shape=jax.ShapeDtypeStruct((NP, HID), jnp.float32),
    )(part, nd, ns, w, b)


def _upd1_body(p_ref, nd_ref, w_ref, b_ref, w1a_ref, w1b_ref, b1_ref,
               p1_ref, p2_ref):
    agg = p_ref[0]
    for cc in range(1, NC):
        agg = agg + p_ref[cc]
    agg = agg * nd_ref[...]
    h = jnp.dot(agg, w_ref[...], preferred_element_type=jnp.float32) + b_ref[...]
    p1_ref[...] = jnp.dot(h, w1a_ref[...], preferred_element_type=jnp.float32) + b1_ref[...]
    p2_ref[...] = jnp.dot(h, w1b_ref[...], preferred_element_type=jnp.float32)


def _upd1(part, nd, w, b, w1a, w1b, b1):
    grid = (NP // _RB,)
    return pl.pallas_call(
        _upd1_body,
        grid=grid,
        in_specs=[
            pl.BlockSpec((NC, _RB, HID), lambda i: (0, i, 0)),
            pl.BlockSpec((_RB, 1), lambda i: (i, 0)),
            pl.BlockSpec((HID, HID), lambda i: (0, 0)),
            pl.BlockSpec((1, HID), lambda i: (0, 0)),
            pl.BlockSpec((HID, HID), lambda i: (0, 0)),
            pl.BlockSpec((HID, HID), lambda i: (0, 0)),
            pl.BlockSpec((1, HID), lambda i: (0, 0)),
        ],
        out_specs=[
            pl.BlockSpec((_RB, HID), lambda i: (i, 0)),
            pl.BlockSpec((_RB, HID), lambda i: (i, 0)),
        ],
        out_shape=[
            jax.ShapeDtypeStruct((NP, HID), jnp.float32),
            jax.ShapeDtypeStruct((NP, HID), jnp.float32),
        ],
    )(part, nd, w, b, w1a, w1b, b1)


_TB = 3200


def _score_body(z_ref, w_ref, b_ref, o_ref):
    o_ref[...] = jnp.dot(z_ref[...], w_ref[...],
                         preferred_element_type=jnp.float32) + b_ref[...]


def _score_tc(z, w2p, b2p):
    grid = (TP // _TB,)
    return pl.pallas_call(
        _score_body,
        grid=grid,
        in_specs=[
            pl.BlockSpec((_TB, HID), lambda i: (i, 0)),
            pl.BlockSpec((HID, HID), lambda i: (0, 0)),
            pl.BlockSpec((1, HID), lambda i: (0, 0)),
        ],
        out_specs=pl.BlockSpec((_TB, HID), lambda i: (i, 0)),
        out_shape=jax.ShapeDtypeStruct((TP, HID), jnp.float32),
    )(z, w2p, b2p)


def kernel(node_feat, edge_index, triplets, emb, conv_W, conv_b,
           mlp_W1, mlp_b1, mlp_W2, mlp_b2):
    prep, segsum, cntsum, mlp_gather = _sc_kernels()

    nf = jnp.pad(node_feat.astype(jnp.int32), (0, NP - N))
    src = edge_index[0].astype(jnp.int32)
    dst = edge_index[1].astype(jnp.int32)
    ts = jnp.pad(triplets[:, 0].astype(jnp.int32), (0, TP - T))
    td = jnp.pad(triplets[:, 2].astype(jnp.int32), (0, TP - T))

    h = prep(nf, emb)
    gidx = jnp.concatenate([jnp.zeros((E,), jnp.int32),
                            jnp.ones((E,), jnp.int32)])
    sidx = jnp.concatenate([src, dst])
    cnt_tab = jnp.zeros((8, HID), jnp.float32).at[0, 0].set(1.0).at[1, 1].set(1.0)
    cnts = cntsum(gidx, sidx, cnt_tab)
    hn0, ns, nd = _norms_tc(cnts, h)
    part0 = segsum(src, dst, hn0)
    hn1 = _upd0(part0, nd, ns, conv_W[0], conv_b[0].reshape(1, HID))
    part1 = segsum(src, dst, hn1)
    p1, p2 = _upd1(part1, nd, conv_W[1], conv_b[1].reshape(1, HID),
                   mlp_W1[:HID], mlp_W1[HID:], mlp_b1.reshape(1, HID))
    z = mlp_gather(ts, td, p1, p2)
    w2p = jnp.pad(mlp_W2, ((0, 0), (0, HID - OUT)))
    b2p = jnp.pad(mlp_b2, (0, HID - OUT)).reshape(1, HID)
    score = _score_tc(z, w2p, b2p)
    return score[:T, :OUT]


# speedup vs baseline: 1.0333x; 1.0333x over previous
import functools

import jax
import jax.numpy as jnp
from jax import lax
from jax.experimental import pallas as pl
from jax.experimental.pallas import tpu as pltpu
from jax.experimental.pallas import tpu_sc as plsc

N = 10000
NP = 10240
E = 320000
HID = 128
T = 100000
TP = 102400
OUT = 2
CHK = 80

NC = 2
NS = 16
NW = NC * NS

EPW = E // NW
ECH = EPW // CHK
HRW = NP // NW
HCH = HRW // CHK
TPW = TP // NW
TCH = TPW // CHK
SRS = NP // NS
SCH = SRS // CHK


def _zero_rows(ref, nrows, width):
    zv = jnp.zeros((16,), jnp.float32)

    def body(r, carry):
        for f in range(width // 16):
            ref[r, pl.ds(f * 16, 16)] = zv
        return carry

    lax.fori_loop(0, nrows, body, 0)


@functools.cache
def _sc_kernels():
    mesh = plsc.VectorSubcoreMesh(core_axis_name="c", subcore_axis_name="s",
                                  num_cores=NC, num_subcores=NS)

    @functools.partial(
        pl.kernel,
        out_type=jax.ShapeDtypeStruct((NP, HID), jnp.float32),
        mesh=mesh,
        scratch_types=[
            pltpu.VMEM((CHK,), jnp.int32),
            pltpu.VMEM((CHK, HID), jnp.float32),
            pltpu.SemaphoreType.DMA,
        ],
    )
    def prep(nf_hbm, emb_hbm, h_hbm, idx_v, rows_v, sem):
        c = lax.axis_index("c")
        s = lax.axis_index("s")
        wid = c * NS + s

        def hbody(k, carry):
            base = (wid * HCH + k) * CHK
            pltpu.sync_copy(nf_hbm.at[pl.ds(base, CHK)], idx_v)
            pltpu.async_copy(emb_hbm.at[idx_v], rows_v, sem).wait()
            pltpu.sync_copy(rows_v, h_hbm.at[pl.ds(base, CHK)])
            return carry

        lax.fori_loop(0, HCH, hbody, 0)

    def make_segsum(num_edges):
      epw = num_edges // NW
      pbuf = 10000
      npass = epw // pbuf
      ech = pbuf // CHK

      @functools.partial(
        pl.kernel,
        out_type=jax.ShapeDtypeStruct((NC, NP, HID), jnp.float32),
        mesh=mesh,
        scratch_types=[
            pltpu.VMEM((pbuf,), jnp.int32),
            pltpu.VMEM((pbuf,), jnp.int32),
            pltpu.VMEM((CHK,), jnp.int32),
            pltpu.VMEM((CHK,), jnp.int32),
            pltpu.VMEM((CHK, HID), jnp.float32),
            pltpu.VMEM((CHK, HID), jnp.float32),
            pltpu.VMEM_SHARED((NP, HID), jnp.float32),
            pltpu.SemaphoreType.DMA,
            pltpu.SemaphoreType.DMA,
            pltpu.SemaphoreType.DMA,
            pltpu.SemaphoreType.DMA,
        ],
      )
      def segsum(src_hbm, dst_hbm, hn_hbm, part_hbm, si_all, di_all,
                 db0, db1, rows0, rows1, acc_sh, semg0, semg1, sems0, sems1):
        c = lax.axis_index("c")
        s = lax.axis_index("s")
        wid = c * NS + s
        base_r = s * SRS
        db = (db0, db1)
        rows = (rows0, rows1)
        semg = (semg0, semg1)
        sems = (sems0, sems1)

        _zero_rows(rows0, CHK, HID)

        def zbody(j, carry):
            pltpu.sync_copy(rows0, acc_sh.at[pl.ds(base_r + j * CHK, CHK)])
            return carry

        lax.fori_loop(0, SCH, zbody, 0)
        plsc.subcore_barrier()

        def load_db(j, b):
            for k in range(CHK // 16):
                db[b][pl.ds(k * 16, 16)] = di_all[pl.ds(j * CHK + k * 16, 16)]

        def start_gather(j, b):
            return pltpu.async_copy(
                hn_hbm.at[si_all.at[pl.ds(j * CHK, CHK)]], rows[b], semg[b])

        def start_scatter(b):
            return pltpu.async_copy(rows[b], acc_sh.at[db[b]], sems[b],
                                    add=True)

        def wait_scatter(b):
            pltpu.make_async_copy(rows[b], acc_sh.at[db[b]], sems[b]).wait()

        def chunk(j, b, first):
            if not first:
                wait_scatter(b)
            load_db(j, b)
            start_gather(j, b).wait()
            start_scatter(b)

        for q in range(npass):
            e0 = wid * epw + q * pbuf
            pltpu.sync_copy(src_hbm.at[pl.ds(e0, pbuf)], si_all)
            pltpu.sync_copy(dst_hbm.at[pl.ds(e0, pbuf)], di_all)

            chunk(0, 0, True)
            chunk(1, 1, True)

            def pbody(p, carry):
                chunk(2 * p, 0, False)
                chunk(2 * p + 1, 1, False)
                return carry

            lax.fori_loop(1, ech // 2, pbody, 0)
            if ech % 2:
                chunk(ech - 1, 0, False)
            wait_scatter(0)
            wait_scatter(1)
        plsc.subcore_barrier()

        def obody(j, carry):
            r0 = base_r + j * CHK
            pltpu.sync_copy(acc_sh.at[pl.ds(r0, CHK)], rows0)
            pltpu.sync_copy(rows0, part_hbm.at[c, pl.ds(r0, CHK)])
            return carry

        lax.fori_loop(0, SCH, obody, 0)

      return segsum

    segsum = make_segsum(E)
    cntsum = make_segsum(2 * E)

    @functools.partial(
        pl.kernel,
        out_type=jax.ShapeDtypeStruct((TP, HID), jnp.float32),
        mesh=mesh,
        scratch_types=[
            pltpu.VMEM((TPW,), jnp.int32),
            pltpu.VMEM((TPW,), jnp.int32),
            pltpu.VMEM((CHK, HID), jnp.float32),
            pltpu.VMEM((CHK, HID), jnp.float32),
            pltpu.VMEM((CHK, HID), jnp.float32),
            pltpu.VMEM((CHK, HID), jnp.float32),
            pltpu.SemaphoreType.DMA,
            pltpu.SemaphoreType.DMA,
            pltpu.SemaphoreType.DMA,
            pltpu.SemaphoreType.DMA,
        ],
    )
    def mlp_gather(ts_hbm, td_hbm, p1_hbm, p2_hbm, z_hbm, ts_all, td_all,
                   g1a, g2a, g1b, g2b, semga, semgb, semwa, semwb):
        c = lax.axis_index("c")
        s = lax.axis_index("s")
        wid = c * NS + s
        g1 = (g1a, g1b)
        g2 = (g2a, g2b)
        semg = (semga, semgb)
        semw = (semwa, semwb)

        pltpu.sync_copy(ts_hbm.at[pl.ds(wid * TPW, TPW)], ts_all)
        pltpu.sync_copy(td_hbm.at[pl.ds(wid * TPW, TPW)], td_all)

        def t0_of(j):
            return wid * TPW + j * CHK

        def chunk(j, b, first):
            if not first:
                pltpu.make_async_copy(g1[b], z_hbm.at[pl.ds(t0_of(j - 2), CHK)],
                                      semw[b]).wait()
            pltpu.async_copy(p1_hbm.at[ts_all.at[pl.ds(j * CHK, CHK)]],
                             g1[b], semg[b])
            pltpu.async_copy(p2_hbm.at[td_all.at[pl.ds(j * CHK, CHK)]],
                             g2[b], semg[b])
            pltpu.make_async_copy(p1_hbm.at[ts_all.at[pl.ds(j * CHK, CHK)]],
                                  g1[b], semg[b]).wait()
            pltpu.make_async_copy(p2_hbm.at[td_all.at[pl.ds(j * CHK, CHK)]],
                                  g2[b], semg[b]).wait()

            def rbody(r, carry2):
                for f in range(HID // 16):
                    sl = pl.ds(f * 16, 16)
                    g1[b][r, sl] = jnp.maximum(g1[b][r, sl] + g2[b][r, sl], 0.0)
                return carry2

            lax.fori_loop(0, CHK, rbody, 0)
            pltpu.async_copy(g1[b], z_hbm.at[pl.ds(t0_of(j), CHK)], semw[b])

        chunk(0, 0, True)
        chunk(1, 1, True)

        def pbody(p, carry):
            chunk(2 * p, 0, False)
            chunk(2 * p + 1, 1, False)
            return carry

        lax.fori_loop(1, TCH // 2, pbody, 0)
        pltpu.make_async_copy(g1[0], z_hbm.at[pl.ds(t0_of(TCH - 2), CHK)],
                              semw[0]).wait()
        pltpu.make_async_copy(g1[1], z_hbm.at[pl.ds(t0_of(TCH - 1), CHK)],
                              semw[1]).wait()

    return prep, segsum, cntsum, mlp_gather


def _norms_body(cnt_ref, h_ref, hn_ref, ns_ref, nd_ref):
    degs = cnt_ref[0, :, 0:1]
    degd = cnt_ref[0, :, 1:2]
    for cc in range(1, NC):
        degs = degs + cnt_ref[cc, :, 0:1]
        degd = degd + cnt_ref[cc, :, 1:2]
    ns = lax.rsqrt(jnp.maximum(degs, 1.0))
    nd = lax.rsqrt(jnp.maximum(degd, 1.0))
    ns_ref[...] = ns
    nd_ref[...] = nd
    hn_ref[...] = h_ref[...] * ns


def _norms_tc(cnts, h):
    return pl.pallas_call(
        _norms_body,
        out_shape=[
            jax.ShapeDtypeStruct((NP, HID), jnp.float32),
            jax.ShapeDtypeStruct((NP, 1), jnp.float32),
            jax.ShapeDtypeStruct((NP, 1), jnp.float32),
        ],
    )(cnts, h)


_RB = 1280


def _upd0_body(p_ref, nd_ref, ns_ref, w_ref, b_ref, o_ref):
    agg = p_ref[0]
    for cc in range(1, NC):
        agg = agg + p_ref[cc]
    agg = agg * nd_ref[...]
    h = jnp.dot(agg, w_ref[...], preferred_element_type=jnp.float32) + b_ref[...]
    o_ref[...] = h * ns_ref[...]


def _upd0(part, nd, ns, w, b):
    grid = (NP // _RB,)
    return pl.pallas_call(
        _upd0_body,
        grid=grid,
        in_specs=[
            pl.BlockSpec((NC, _RB, HID), lambda i: (0, i, 0)),
            pl.BlockSpec((_RB, 1), lambda i: (i, 0)),
            pl.BlockSpec((_RB, 1), lambda i: (i, 0)),
            pl.BlockSpec((HID, HID), lambda i: (0, 0)),
            pl.BlockSpec((1, HID), lambda i: (0, 0)),
        ],
        out_specs=pl.BlockSpec((_RB, HID), lambda i: (i, 0)),
        out_shape=jax.ShapeDtypeStruct((NP, HID), jnp.float32),
    )(part, nd, ns, w, b)


def _upd1_body(p_ref, nd_ref, w_ref, b_ref, w1a_ref, w1b_ref, b1_ref,
               p1_ref, p2_ref):
    agg = p_ref[0]
    for cc in range(1, NC):
        agg = agg + p_ref[cc]
    agg = agg * nd_ref[...]
    h = jnp.dot(agg, w_ref[...], preferred_element_type=jnp.float32) + b_ref[...]
    p1_ref[...] = jnp.dot(h, w1a_ref[...], preferred_element_type=jnp.float32) + b1_ref[...]
    p2_ref[...] = jnp.dot(h, w1b_ref[...], preferred_element_type=jnp.float32)


def _upd1(part, nd, w, b, w1a, w1b, b1):
    grid = (NP // _RB,)
    return pl.pallas_call(
        _upd1_body,
        grid=grid,
        in_specs=[
            pl.BlockSpec((NC, _RB, HID), lambda i: (0, i, 0)),
            pl.BlockSpec((_RB, 1), lambda i: (i, 0)),
            pl.BlockSpec((HID, HID), lambda i: (0, 0)),
            pl.BlockSpec((1, HID), lambda i: (0, 0)),
            pl.BlockSpec((HID, HID), lambda i: (0, 0)),
            pl.BlockSpec((HID, HID), lambda i: (0, 0)),
            pl.BlockSpec((1, HID), lambda i: (0, 0)),
        ],
        out_specs=[
            pl.BlockSpec((_RB, HID), lambda i: (i, 0)),
            pl.BlockSpec((_RB, HID), lambda i: (i, 0)),
        ],
        out_shape=[
            jax.ShapeDtypeStruct((NP, HID), jnp.float32),
            jax.ShapeDtypeStruct((NP, HID), jnp.float32),
        ],
    )(part, nd, w, b, w1a, w1b, b1)


_TB = 3200


def _score_body(z_ref, w_ref, b_ref, o_ref):
    o_ref[...] = jnp.dot(z_ref[...], w_ref[...],
                         preferred_element_type=jnp.float32) + b_ref[...]


def _score_tc(z, w2p, b2p):
    grid = (TP // _TB,)
    return pl.pallas_call(
        _score_body,
        grid=grid,
        in_specs=[
            pl.BlockSpec((_TB, HID), lambda i: (i, 0)),
            pl.BlockSpec((HID, HID), lambda i: (0, 0)),
            pl.BlockSpec((1, HID), lambda i: (0, 0)),
        ],
        out_specs=pl.BlockSpec((_TB, HID), lambda i: (i, 0)),
        out_shape=jax.ShapeDtypeStruct((TP, HID), jnp.float32),
    )(z, w2p, b2p)


def kernel(node_feat, edge_index, triplets, emb, conv_W, conv_b,
           mlp_W1, mlp_b1, mlp_W2, mlp_b2):
    prep, segsum, cntsum, mlp_gather = _sc_kernels()

    nf = jnp.pad(node_feat.astype(jnp.int32), (0, NP - N))
    src = edge_index[0].astype(jnp.int32)
    dst = edge_index[1].astype(jnp.int32)
    ts = jnp.pad(triplets[:, 0].astype(jnp.int32), (0, TP - T))
    td = jnp.pad(triplets[:, 2].astype(jnp.int32), (0, TP - T))

    h = prep(nf, emb)
    gidx = jnp.concatenate([jnp.zeros((E,), jnp.int32),
                            jnp.ones((E,), jnp.int32)])
    sidx = jnp.concatenate([src, dst])
    cnt_tab = jnp.zeros((8, HID), jnp.float32).at[0, 0].set(1.0).at[1, 1].set(1.0)
    cnts = cntsum(gidx, sidx, cnt_tab)
    hn0, ns, nd = _norms_tc(cnts, h)
    part0 = segsum(src, dst, hn0)
    hn1 = _upd0(part0, nd, ns, conv_W[0], conv_b[0].reshape(1, HID))
    part1 = segsum(src, dst, hn1)
    p1, p2 = _upd1(part1, nd, conv_W[1], conv_b[1].reshape(1, HID),
                   mlp_W1[:HID], mlp_W1[HID:], mlp_b1.reshape(1, HID))
    z = mlp_gather(ts, td, p1, p2)
    w2p = jnp.pad(mlp_W2, ((0, 0), (0, HID - OUT)))
    b2p = jnp.pad(mlp_b2, (0, HID - OUT)).reshape(1, HID)
    score = _score_tc(z, w2p, b2p)
    return score[:T, :OUT]


# speedup vs baseline: 7.8857x; 7.6318x over previous
import functools

import jax
import jax.numpy as jnp
from jax import lax
from jax.experimental import pallas as pl
from jax.experimental.pallas import tpu as pltpu
from jax.experimental.pallas import tpu_sc as plsc

N = 10000
NP = 10240
E = 320000
EP = 327680
HID = 128
T = 100000
TP = 102400
OUT = 2
CHK = 128
PCHK = 80

NC = 2
NS = 16
NW = NC * NS

HRW = NP // NW
HCH = HRW // PCHK
TPW = TP // NW
TCH = TPW // CHK
SRS = NP // NS
SCH = SRS // CHK


def _zero_rows(ref, nrows, width):
    zv = jnp.zeros((16,), jnp.float32)

    def body(r, carry):
        for f in range(width // 16):
            ref[r, pl.ds(f * 16, 16)] = zv
        return carry

    lax.fori_loop(0, nrows, body, 0)


@functools.cache
def _sc_kernels():
    mesh = plsc.VectorSubcoreMesh(core_axis_name="c", subcore_axis_name="s",
                                  num_cores=NC, num_subcores=NS)

    @functools.partial(
        pl.kernel,
        out_type=jax.ShapeDtypeStruct((NP, HID), jnp.float32),
        mesh=mesh,
        scratch_types=[
            pltpu.VMEM((PCHK,), jnp.int32),
            pltpu.VMEM((PCHK, HID), jnp.float32),
            pltpu.SemaphoreType.DMA,
        ],
    )
    def prep(nf_hbm, emb_hbm, h_hbm, idx_v, rows_v, sem):
        c = lax.axis_index("c")
        s = lax.axis_index("s")
        wid = c * NS + s

        def hbody(k, carry):
            base = (wid * HCH + k) * PCHK
            pltpu.sync_copy(nf_hbm.at[pl.ds(base, PCHK)], idx_v)
            pltpu.async_copy(emb_hbm.at[idx_v], rows_v, sem).wait()
            pltpu.sync_copy(rows_v, h_hbm.at[pl.ds(base, PCHK)])
            return carry

        lax.fori_loop(0, HCH, hbody, 0)

    def make_segsum(num_edges):
      epw = num_edges // NW
      pbuf = 5120
      npass = epw // pbuf
      ech = pbuf // CHK

      @functools.partial(
        pl.kernel,
        out_type=jax.ShapeDtypeStruct((NC, NP, HID), jnp.float32),
        mesh=mesh,
        scratch_types=[
            pltpu.VMEM((pbuf,), jnp.int32),
            pltpu.VMEM((pbuf,), jnp.int32),
            pltpu.VMEM((CHK,), jnp.int32),
            pltpu.VMEM((CHK,), jnp.int32),
            pltpu.VMEM((CHK, HID), jnp.float32),
            pltpu.VMEM((CHK, HID), jnp.float32),
            pltpu.VMEM_SHARED((NP, HID), jnp.float32),
            pltpu.SemaphoreType.DMA,
            pltpu.SemaphoreType.DMA,
            pltpu.SemaphoreType.DMA,
            pltpu.SemaphoreType.DMA,
        ],
      )
      def segsum(src_hbm, dst_hbm, hn_hbm, part_hbm, si_all, di_all,
                 db0, db1, rows0, rows1, acc_sh, semg0, semg1, sems0, sems1):
        c = lax.axis_index("c")
        s = lax.axis_index("s")
        wid = c * NS + s
        base_r = s * SRS
        db = (db0, db1)
        rows = (rows0, rows1)
        semg = (semg0, semg1)
        sems = (sems0, sems1)

        _zero_rows(rows0, CHK, HID)

        def zbody(j, carry):
            pltpu.sync_copy(rows0, acc_sh.at[pl.ds(base_r + j * CHK, CHK)])
            return carry

        lax.fori_loop(0, SCH, zbody, 0)
        plsc.subcore_barrier()

        def load_db(j, b):
            for k in range(CHK // 16):
                db[b][pl.ds(k * 16, 16)] = di_all[pl.ds(j * CHK + k * 16, 16)]

        def start_gather(j, b):
            return pltpu.async_copy(
                hn_hbm.at[si_all.at[pl.ds(j * CHK, CHK)]], rows[b], semg[b])

        def start_scatter(b):
            return pltpu.async_copy(rows[b], acc_sh.at[db[b]], sems[b],
                                    add=True)

        def wait_scatter(b):
            pltpu.make_async_copy(rows[b], acc_sh.at[db[b]], sems[b]).wait()

        def chunk(j, b, first):
            if not first:
                wait_scatter(b)
            load_db(j, b)
            start_gather(j, b).wait()
            start_scatter(b)

        for q in range(npass):
            e0 = wid * epw + q * pbuf
            pltpu.sync_copy(src_hbm.at[pl.ds(e0, pbuf)], si_all)
            pltpu.sync_copy(dst_hbm.at[pl.ds(e0, pbuf)], di_all)

            chunk(0, 0, True)
            chunk(1, 1, True)

            def pbody(p, carry):
                chunk(2 * p, 0, False)
                chunk(2 * p + 1, 1, False)
                return carry

            lax.fori_loop(1, ech // 2, pbody, 0)
            if ech % 2:
                chunk(ech - 1, 0, False)
            wait_scatter(0)
            wait_scatter(1)
        plsc.subcore_barrier()

        def obody(j, carry):
            r0 = base_r + j * CHK
            pltpu.sync_copy(acc_sh.at[pl.ds(r0, CHK)], rows0)
            pltpu.sync_copy(rows0, part_hbm.at[c, pl.ds(r0, CHK)])
            return carry

        lax.fori_loop(0, SCH, obody, 0)

      return segsum

    segsum = make_segsum(EP)
    cntsum = make_segsum(2 * EP)

    @functools.partial(
        pl.kernel,
        out_type=jax.ShapeDtypeStruct((TP, HID), jnp.float32),
        mesh=mesh,
        scratch_types=[
            pltpu.VMEM((TPW,), jnp.int32),
            pltpu.VMEM((TPW,), jnp.int32),
            pltpu.VMEM((CHK, HID), jnp.float32),
            pltpu.VMEM((CHK, HID), jnp.float32),
            pltpu.VMEM((CHK, HID), jnp.float32),
            pltpu.VMEM((CHK, HID), jnp.float32),
            pltpu.SemaphoreType.DMA,
            pltpu.SemaphoreType.DMA,
            pltpu.SemaphoreType.DMA,
            pltpu.SemaphoreType.DMA,
        ],
    )
    def mlp_gather(ts_hbm, td_hbm, p1_hbm, p2_hbm, z_hbm, ts_all, td_all,
                   g1a, g2a, g1b, g2b, semga, semgb, semwa, semwb):
        c = lax.axis_index("c")
        s = lax.axis_index("s")
        wid = c * NS + s
        g1 = (g1a, g1b)
        g2 = (g2a, g2b)
        semg = (semga, semgb)
        semw = (semwa, semwb)

        pltpu.sync_copy(ts_hbm.at[pl.ds(wid * TPW, TPW)], ts_all)
        pltpu.sync_copy(td_hbm.at[pl.ds(wid * TPW, TPW)], td_all)

        def t0_of(j):
            return wid * TPW + j * CHK

        def chunk(j, b, first):
            if not first:
                pltpu.make_async_copy(g1[b], z_hbm.at[pl.ds(t0_of(j - 2), CHK)],
                                      semw[b]).wait()
            pltpu.async_copy(p1_hbm.at[ts_all.at[pl.ds(j * CHK, CHK)]],
                             g1[b], semg[b])
            pltpu.async_copy(p2_hbm.at[td_all.at[pl.ds(j * CHK, CHK)]],
                             g2[b], semg[b])
            pltpu.make_async_copy(p1_hbm.at[ts_all.at[pl.ds(j * CHK, CHK)]],
                                  g1[b], semg[b]).wait()
            pltpu.make_async_copy(p2_hbm.at[td_all.at[pl.ds(j * CHK, CHK)]],
                                  g2[b], semg[b]).wait()

            def rbody(r, carry2):
                for f in range(HID // 16):
                    sl = pl.ds(f * 16, 16)
                    g1[b][r, sl] = jnp.maximum(g1[b][r, sl] + g2[b][r, sl], 0.0)
                return carry2

            lax.fori_loop(0, CHK, rbody, 0)
            pltpu.async_copy(g1[b], z_hbm.at[pl.ds(t0_of(j), CHK)], semw[b])

        chunk(0, 0, True)
        chunk(1, 1, True)

        def pbody(p, carry):
            chunk(2 * p, 0, False)
            chunk(2 * p + 1, 1, False)
            return carry

        lax.fori_loop(1, TCH // 2, pbody, 0)
        if TCH % 2:
            chunk(TCH - 1, 0, False)
            last0, last1 = TCH - 1, TCH - 2
        else:
            last0, last1 = TCH - 2, TCH - 1
        pltpu.make_async_copy(g1[0], z_hbm.at[pl.ds(t0_of(last0), CHK)],
                              semw[0]).wait()
        pltpu.make_async_copy(g1[1], z_hbm.at[pl.ds(t0_of(last1), CHK)],
                              semw[1]).wait()

    return prep, segsum, cntsum, mlp_gather


def _norms_body(cnt_ref, h_ref, hn_ref, ns_ref, nd_ref):
    degs = cnt_ref[0, :, 0:1]
    degd = cnt_ref[0, :, 1:2]
    for cc in range(1, NC):
        degs = degs + cnt_ref[cc, :, 0:1]
        degd = degd + cnt_ref[cc, :, 1:2]
    ns = lax.rsqrt(jnp.maximum(degs, 1.0))
    nd = lax.rsqrt(jnp.maximum(degd, 1.0))
    ns_ref[...] = ns
    nd_ref[...] = nd
    hn_ref[...] = h_ref[...] * ns


def _norms_tc(cnts, h):
    return pl.pallas_call(
        _norms_body,
        out_shape=[
            jax.ShapeDtypeStruct((NP, HID), jnp.float32),
            jax.ShapeDtypeStruct((NP, 1), jnp.float32),
            jax.ShapeDtypeStruct((NP, 1), jnp.float32),
        ],
    )(cnts, h)


_RB = 1280


def _upd0_body(p_ref, nd_ref, ns_ref, w_ref, b_ref, o_ref):
    agg = p_ref[0]
    for cc in range(1, NC):
        agg = agg + p_ref[cc]
    agg = agg * nd_ref[...]
    h = jnp.dot(agg, w_ref[...], preferred_element_type=jnp.float32) + b_ref[...]
    o_ref[...] = h * ns_ref[...]


def _upd0(part, nd, ns, w, b):
    grid = (NP // _RB,)
    return pl.pallas_call(
        _upd0_body,
        grid=grid,
        in_specs=[
            pl.BlockSpec((NC, _RB, HID), lambda i: (0, i, 0)),
            pl.BlockSpec((_RB, 1), lambda i: (i, 0)),
            pl.BlockSpec((_RB, 1), lambda i: (i, 0)),
            pl.BlockSpec((HID, HID), lambda i: (0, 0)),
            pl.BlockSpec((1, HID), lambda i: (0, 0)),
        ],
        out_specs=pl.BlockSpec((_RB, HID), lambda i: (i, 0)),
        out_shape=jax.ShapeDtypeStruct((NP, HID), jnp.float32),
    )(part, nd, ns, w, b)


def _upd1_body(p_ref, nd_ref, w_ref, b_ref, w1a_ref, w1b_ref, b1_ref,
               p1_ref, p2_ref):
    agg = p_ref[0]
    for cc in range(1, NC):
        agg = agg + p_ref[cc]
    agg = agg * nd_ref[...]
    h = jnp.dot(agg, w_ref[...], preferred_element_type=jnp.float32) + b_ref[...]
    p1_ref[...] = jnp.dot(h, w1a_ref[...], preferred_element_type=jnp.float32) + b1_ref[...]
    p2_ref[...] = jnp.dot(h, w1b_ref[...], preferred_element_type=jnp.float32)


def _upd1(part, nd, w, b, w1a, w1b, b1):
    grid = (NP // _RB,)
    return pl.pallas_call(
        _upd1_body,
        grid=grid,
        in_specs=[
            pl.BlockSpec((NC, _RB, HID), lambda i: (0, i, 0)),
            pl.BlockSpec((_RB, 1), lambda i: (i, 0)),
            pl.BlockSpec((HID, HID), lambda i: (0, 0)),
            pl.BlockSpec((1, HID), lambda i: (0, 0)),
            pl.BlockSpec((HID, HID), lambda i: (0, 0)),
            pl.BlockSpec((HID, HID), lambda i: (0, 0)),
            pl.BlockSpec((1, HID), lambda i: (0, 0)),
        ],
        out_specs=[
            pl.BlockSpec((_RB, HID), lambda i: (i, 0)),
            pl.BlockSpec((_RB, HID), lambda i: (i, 0)),
        ],
        out_shape=[
            jax.ShapeDtypeStruct((NP, HID), jnp.float32),
            jax.ShapeDtypeStruct((NP, HID), jnp.float32),
        ],
    )(part, nd, w, b, w1a, w1b, b1)


_TB = 3200


def _score_body(z_ref, w_ref, b_ref, o_ref):
    o_ref[...] = jnp.dot(z_ref[...], w_ref[...],
                         preferred_element_type=jnp.float32) + b_ref[...]


def _score_tc(z, w2, b2):
    grid = (TP // _TB,)
    return pl.pallas_call(
        _score_body,
        grid=grid,
        in_specs=[
            pl.BlockSpec((_TB, HID), lambda i: (i, 0)),
            pl.BlockSpec((HID, OUT), lambda i: (0, 0)),
            pl.BlockSpec((1, OUT), lambda i: (0, 0)),
        ],
        out_specs=pl.BlockSpec((_TB, OUT), lambda i: (i, 0)),
        out_shape=jax.ShapeDtypeStruct((TP, OUT), jnp.float32),
    )(z, w2, b2)


def kernel(node_feat, edge_index, triplets, emb, conv_W, conv_b,
           mlp_W1, mlp_b1, mlp_W2, mlp_b2):
    prep, segsum, cntsum, mlp_gather = _sc_kernels()

    nf = jnp.pad(node_feat.astype(jnp.int32), (0, NP - N))
    src = jnp.pad(edge_index[0].astype(jnp.int32), (0, EP - E))
    dst = jnp.pad(edge_index[1].astype(jnp.int32), (0, EP - E),
                  constant_values=N)
    ts = jnp.pad(triplets[:, 0].astype(jnp.int32), (0, TP - T))
    td = jnp.pad(triplets[:, 2].astype(jnp.int32), (0, TP - T))

    h = prep(nf, emb)
    lanes = jnp.arange(EP, dtype=jnp.int32) % 64
    gidx = jnp.concatenate([lanes, 64 + lanes])
    srcc = jnp.pad(edge_index[0].astype(jnp.int32), (0, EP - E),
                   constant_values=N)
    sidx = jnp.concatenate([srcc, dst])
    cnt_tab = jnp.concatenate([
        jnp.tile(jax.nn.one_hot(0, HID, dtype=jnp.float32)[None], (64, 1)),
        jnp.tile(jax.nn.one_hot(1, HID, dtype=jnp.float32)[None], (64, 1)),
    ])
    cnts = cntsum(gidx, sidx, cnt_tab)
    hn0, ns, nd = _norms_tc(cnts, h)
    part0 = segsum(src, dst, hn0)
    hn1 = _upd0(part0, nd, ns, conv_W[0], conv_b[0].reshape(1, HID))
    part1 = segsum(src, dst, hn1)
    p1, p2 = _upd1(part1, nd, conv_W[1], conv_b[1].reshape(1, HID),
                   mlp_W1[:HID], mlp_W1[HID:], mlp_b1.reshape(1, HID))
    z = mlp_gather(ts, td, p1, p2)
    score = _score_tc(z, mlp_W2, mlp_b2.reshape(1, OUT))
    return score[:T]


# speedup vs baseline: 7.8886x; 1.0004x over previous
import functools

import jax
import jax.numpy as jnp
from jax import lax
from jax.experimental import pallas as pl
from jax.experimental.pallas import tpu as pltpu
from jax.experimental.pallas import tpu_sc as plsc

N = 10000
NP = 10240
E = 320000
EP = 327680
HID = 128
T = 100000
TP = 102400
OUT = 2
CHK = 128
PCHK = 80

NC = 2
NS = 16
NW = NC * NS

HRW = NP // NW
HCH = HRW // PCHK
TPW = TP // NW
TCH = TPW // CHK
SRS = NP // NS
SCH = SRS // CHK


def _zero_rows(ref, nrows, width):
    zv = jnp.zeros((16,), jnp.float32)

    def body(r, carry):
        for f in range(width // 16):
            ref[r, pl.ds(f * 16, 16)] = zv
        return carry

    lax.fori_loop(0, nrows, body, 0)


@functools.cache
def _sc_kernels():
    mesh = plsc.VectorSubcoreMesh(core_axis_name="c", subcore_axis_name="s",
                                  num_cores=NC, num_subcores=NS)

    @functools.partial(
        pl.kernel,
        out_type=jax.ShapeDtypeStruct((NP, HID), jnp.float32),
        mesh=mesh,
        scratch_types=[
            pltpu.VMEM((PCHK,), jnp.int32),
            pltpu.VMEM((PCHK, HID), jnp.float32),
            pltpu.SemaphoreType.DMA,
        ],
    )
    def prep(nf_hbm, emb_hbm, h_hbm, idx_v, rows_v, sem):
        c = lax.axis_index("c")
        s = lax.axis_index("s")
        wid = c * NS + s

        def hbody(k, carry):
            base = (wid * HCH + k) * PCHK
            pltpu.sync_copy(nf_hbm.at[pl.ds(base, PCHK)], idx_v)
            pltpu.async_copy(emb_hbm.at[idx_v], rows_v, sem).wait()
            pltpu.sync_copy(rows_v, h_hbm.at[pl.ds(base, PCHK)])
            return carry

        lax.fori_loop(0, HCH, hbody, 0)

    def make_segsum(num_edges):
      epw = num_edges // NW
      pbuf = 5120
      npass = epw // pbuf
      ech = pbuf // CHK

      @functools.partial(
        pl.kernel,
        out_type=jax.ShapeDtypeStruct((NC, NP, HID), jnp.float32),
        mesh=mesh,
        scratch_types=[
            pltpu.VMEM((pbuf,), jnp.int32),
            pltpu.VMEM((pbuf,), jnp.int32),
            pltpu.VMEM((CHK,), jnp.int32),
            pltpu.VMEM((CHK,), jnp.int32),
            pltpu.VMEM((CHK, HID), jnp.float32),
            pltpu.VMEM((CHK, HID), jnp.float32),
            pltpu.VMEM_SHARED((NP, HID), jnp.float32),
            pltpu.SemaphoreType.DMA,
            pltpu.SemaphoreType.DMA,
            pltpu.SemaphoreType.DMA,
            pltpu.SemaphoreType.DMA,
        ],
      )
      def segsum(src_hbm, dst_hbm, hn_hbm, part_hbm, si_all, di_all,
                 db0, db1, rows0, rows1, acc_sh, semg0, semg1, sems0, sems1):
        c = lax.axis_index("c")
        s = lax.axis_index("s")
        wid = c * NS + s
        base_r = s * SRS
        db = (db0, db1)
        rows = (rows0, rows1)
        semg = (semg0, semg1)
        sems = (sems0, sems1)

        _zero_rows(rows0, CHK, HID)

        def zbody(j, carry):
            pltpu.sync_copy(rows0, acc_sh.at[pl.ds(base_r + j * CHK, CHK)])
            return carry

        lax.fori_loop(0, SCH, zbody, 0)
        plsc.subcore_barrier()

        def load_db(j, b):
            for k in range(CHK // 16):
                db[b][pl.ds(k * 16, 16)] = di_all[pl.ds(j * CHK + k * 16, 16)]

        def start_gather(j, b):
            return pltpu.async_copy(
                hn_hbm.at[si_all.at[pl.ds(j * CHK, CHK)]], rows[b], semg[b])

        def start_scatter(b):
            return pltpu.async_copy(rows[b], acc_sh.at[db[b]], sems[b],
                                    add=True)

        def wait_scatter(b):
            pltpu.make_async_copy(rows[b], acc_sh.at[db[b]], sems[b]).wait()

        def chunk(j, b, first):
            if not first:
                wait_scatter(b)
            load_db(j, b)
            start_gather(j, b).wait()
            start_scatter(b)

        for q in range(npass):
            e0 = wid * epw + q * pbuf
            pltpu.sync_copy(src_hbm.at[pl.ds(e0, pbuf)], si_all)
            pltpu.sync_copy(dst_hbm.at[pl.ds(e0, pbuf)], di_all)

            chunk(0, 0, True)
            chunk(1, 1, True)

            def pbody(p, carry):
                chunk(2 * p, 0, False)
                chunk(2 * p + 1, 1, False)
                return carry

            lax.fori_loop(1, ech // 2, pbody, 0)
            if ech % 2:
                chunk(ech - 1, 0, False)
            wait_scatter(0)
            wait_scatter(1)
        plsc.subcore_barrier()

        def obody(j, carry):
            r0 = base_r + j * CHK
            pltpu.sync_copy(acc_sh.at[pl.ds(r0, CHK)], rows0)
            pltpu.sync_copy(rows0, part_hbm.at[c, pl.ds(r0, CHK)])
            return carry

        lax.fori_loop(0, SCH, obody, 0)

      return segsum

    segsum = make_segsum(EP)

    def make_cntsum(num_edges):
      epw = num_edges // NW
      pbuf = 5120
      npass = epw // pbuf
      ech = pbuf // CHK

      @functools.partial(
        pl.kernel,
        out_type=jax.ShapeDtypeStruct((NC, NP, HID), jnp.float32),
        mesh=mesh,
        scratch_types=[
            pltpu.VMEM((pbuf,), jnp.int32),
            pltpu.VMEM((CHK,), jnp.int32),
            pltpu.VMEM((CHK,), jnp.int32),
            pltpu.VMEM((CHK,), jnp.int32),
            pltpu.VMEM((CHK, HID), jnp.float32),
            pltpu.VMEM((CHK, HID), jnp.float32),
            pltpu.VMEM_SHARED((NP, HID), jnp.float32),
            pltpu.SemaphoreType.DMA,
            pltpu.SemaphoreType.DMA,
            pltpu.SemaphoreType.DMA,
        ],
      )
      def cntsum(sidx_hbm, tab_hbm, part_hbm, di_all, gi_v, db0, db1,
                 rows0, rows1, acc_sh, semg, sems0, sems1):
        c = lax.axis_index("c")
        s = lax.axis_index("s")
        wid = c * NS + s
        base_r = s * SRS
        db = (db0, db1)
        rows = (rows0, rows1)
        sems = (sems0, sems1)
        io16 = lax.iota(jnp.int32, 16)

        for k in range(CHK // 16):
            gi_v[pl.ds(k * 16, 16)] = 128 + ((io16 + 16 * k) % 64)
        pltpu.async_copy(tab_hbm.at[gi_v], rows0, semg).wait()

        def zbody(j, carry):
            pltpu.sync_copy(rows0, acc_sh.at[pl.ds(base_r + j * CHK, CHK)])
            return carry

        lax.fori_loop(0, SCH, zbody, 0)

        for k in range(CHK // 16):
            gi_v[pl.ds(k * 16, 16)] = 64 * c + ((io16 + 16 * k) % 64)
        pltpu.async_copy(tab_hbm.at[gi_v], rows0, semg).wait()
        pltpu.async_copy(tab_hbm.at[gi_v], rows1, semg).wait()
        plsc.subcore_barrier()

        def load_db(j, b):
            for k in range(CHK // 16):
                db[b][pl.ds(k * 16, 16)] = di_all[pl.ds(j * CHK + k * 16, 16)]

        def chunk(j, b, first):
            if not first:
                pltpu.make_async_copy(rows[b], acc_sh.at[db[b]],
                                      sems[b]).wait()
            load_db(j, b)
            pltpu.async_copy(rows[b], acc_sh.at[db[b]], sems[b], add=True)

        for q in range(npass):
            e0 = wid * epw + q * pbuf
            pltpu.sync_copy(sidx_hbm.at[pl.ds(e0, pbuf)], di_all)
            chunk(0, 0, True)
            chunk(1, 1, True)

            def pbody(p, carry):
                chunk(2 * p, 0, False)
                chunk(2 * p + 1, 1, False)
                return carry

            lax.fori_loop(1, ech // 2, pbody, 0)
            if ech % 2:
                chunk(ech - 1, 0, False)
            pltpu.make_async_copy(rows[0], acc_sh.at[db[0]], sems[0]).wait()
            pltpu.make_async_copy(rows[1], acc_sh.at[db[1]], sems[1]).wait()
        plsc.subcore_barrier()

        def obody(j, carry):
            r0 = base_r + j * CHK
            pltpu.sync_copy(acc_sh.at[pl.ds(r0, CHK)], rows0)
            pltpu.sync_copy(rows0, part_hbm.at[c, pl.ds(r0, CHK)])
            return carry

        lax.fori_loop(0, SCH, obody, 0)

      return cntsum

    cntsum = make_cntsum(2 * EP)

    @functools.partial(
        pl.kernel,
        out_type=jax.ShapeDtypeStruct((TP, HID), jnp.float32),
        mesh=mesh,
        scratch_types=[
            pltpu.VMEM((TPW,), jnp.int32),
            pltpu.VMEM((TPW,), jnp.int32),
            pltpu.VMEM((CHK, HID), jnp.float32),
            pltpu.VMEM((CHK, HID), jnp.float32),
            pltpu.VMEM((CHK, HID), jnp.float32),
            pltpu.VMEM((CHK, HID), jnp.float32),
            pltpu.SemaphoreType.DMA,
            pltpu.SemaphoreType.DMA,
            pltpu.SemaphoreType.DMA,
            pltpu.SemaphoreType.DMA,
        ],
    )
    def mlp_gather(ts_hbm, td_hbm, p1_hbm, p2_hbm, z_hbm, ts_all, td_all,
                   g1a, g2a, g1b, g2b, semga, semgb, semwa, semwb):
        c = lax.axis_index("c")
        s = lax.axis_index("s")
        wid = c * NS + s
        g1 = (g1a, g1b)
        g2 = (g2a, g2b)
        semg = (semga, semgb)
        semw = (semwa, semwb)

        pltpu.sync_copy(ts_hbm.at[pl.ds(wid * TPW, TPW)], ts_all)
        pltpu.sync_copy(td_hbm.at[pl.ds(wid * TPW, TPW)], td_all)

        def t0_of(j):
            return wid * TPW + j * CHK

        def chunk(j, b, first):
            if not first:
                pltpu.make_async_copy(g1[b], z_hbm.at[pl.ds(t0_of(j - 2), CHK)],
                                      semw[b]).wait()
            pltpu.async_copy(p1_hbm.at[ts_all.at[pl.ds(j * CHK, CHK)]],
                             g1[b], semg[b])
            pltpu.async_copy(p2_hbm.at[td_all.at[pl.ds(j * CHK, CHK)]],
                             g2[b], semg[b])
            pltpu.make_async_copy(p1_hbm.at[ts_all.at[pl.ds(j * CHK, CHK)]],
                                  g1[b], semg[b]).wait()
            pltpu.make_async_copy(p2_hbm.at[td_all.at[pl.ds(j * CHK, CHK)]],
                                  g2[b], semg[b]).wait()

            def rbody(r, carry2):
                for f in range(HID // 16):
                    sl = pl.ds(f * 16, 16)
                    g1[b][r, sl] = jnp.maximum(g1[b][r, sl] + g2[b][r, sl], 0.0)
                return carry2

            lax.fori_loop(0, CHK, rbody, 0)
            pltpu.async_copy(g1[b], z_hbm.at[pl.ds(t0_of(j), CHK)], semw[b])

        chunk(0, 0, True)
        chunk(1, 1, True)

        def pbody(p, carry):
            chunk(2 * p, 0, False)
            chunk(2 * p + 1, 1, False)
            return carry

        lax.fori_loop(1, TCH // 2, pbody, 0)
        if TCH % 2:
            chunk(TCH - 1, 0, False)
            last0, last1 = TCH - 1, TCH - 2
        else:
            last0, last1 = TCH - 2, TCH - 1
        pltpu.make_async_copy(g1[0], z_hbm.at[pl.ds(t0_of(last0), CHK)],
                              semw[0]).wait()
        pltpu.make_async_copy(g1[1], z_hbm.at[pl.ds(t0_of(last1), CHK)],
                              semw[1]).wait()

    return prep, segsum, cntsum, mlp_gather


def _norms_body(cnt_ref, h_ref, hn_ref, ns_ref, nd_ref):
    degs = cnt_ref[0, :, 0:1]
    degd = cnt_ref[0, :, 1:2]
    for cc in range(1, NC):
        degs = degs + cnt_ref[cc, :, 0:1]
        degd = degd + cnt_ref[cc, :, 1:2]
    ns = lax.rsqrt(jnp.maximum(degs, 1.0))
    nd = lax.rsqrt(jnp.maximum(degd, 1.0))
    ns_ref[...] = ns
    nd_ref[...] = nd
    hn_ref[...] = h_ref[...] * ns


def _norms_tc(cnts, h):
    return pl.pallas_call(
        _norms_body,
        out_shape=[
            jax.ShapeDtypeStruct((NP, HID), jnp.float32),
            jax.ShapeDtypeStruct((NP, 1), jnp.float32),
            jax.ShapeDtypeStruct((NP, 1), jnp.float32),
        ],
    )(cnts, h)


_RB = 1280


def _upd0_body(p_ref, nd_ref, ns_ref, w_ref, b_ref, o_ref):
    agg = p_ref[0]
    for cc in range(1, NC):
        agg = agg + p_ref[cc]
    agg = agg * nd_ref[...]
    h = jnp.dot(agg, w_ref[...], preferred_element_type=jnp.float32) + b_ref[...]
    o_ref[...] = h * ns_ref[...]


def _upd0(part, nd, ns, w, b):
    grid = (NP // _RB,)
    return pl.pallas_call(
        _upd0_body,
        grid=grid,
        in_specs=[
            pl.BlockSpec((NC, _RB, HID), lambda i: (0, i, 0)),
            pl.BlockSpec((_RB, 1), lambda i: (i, 0)),
            pl.BlockSpec((_RB, 1), lambda i: (i, 0)),
            pl.BlockSpec((HID, HID), lambda i: (0, 0)),
            pl.BlockSpec((1, HID), lambda i: (0, 0)),
        ],
        out_specs=pl.BlockSpec((_RB, HID), lambda i: (i, 0)),
        out_shape=jax.ShapeDtypeStruct((NP, HID), jnp.float32),
    )(part, nd, ns, w, b)


def _upd1_body(p_ref, nd_ref, w_ref, b_ref, w1a_ref, w1b_ref, b1_ref,
               p1_ref, p2_ref):
    agg = p_ref[0]
    for cc in range(1, NC):
        agg = agg + p_ref[cc]
    agg = agg * nd_ref[...]
    h = jnp.dot(agg, w_ref[...], preferred_element_type=jnp.float32) + b_ref[...]
    p1_ref[...] = jnp.dot(h, w1a_ref[...], preferred_element_type=jnp.float32) + b1_ref[...]
    p2_ref[...] = jnp.dot(h, w1b_ref[...], preferred_element_type=jnp.float32)


def _upd1(part, nd, w, b, w1a, w1b, b1):
    grid = (NP // _RB,)
    return pl.pallas_call(
        _upd1_body,
        grid=grid,
        in_specs=[
            pl.BlockSpec((NC, _RB, HID), lambda i: (0, i, 0)),
            pl.BlockSpec((_RB, 1), lambda i: (i, 0)),
            pl.BlockSpec((HID, HID), lambda i: (0, 0)),
            pl.BlockSpec((1, HID), lambda i: (0, 0)),
            pl.BlockSpec((HID, HID), lambda i: (0, 0)),
            pl.BlockSpec((HID, HID), lambda i: (0, 0)),
            pl.BlockSpec((1, HID), lambda i: (0, 0)),
        ],
        out_specs=[
            pl.BlockSpec((_RB, HID), lambda i: (i, 0)),
            pl.BlockSpec((_RB, HID), lambda i: (i, 0)),
        ],
        out_shape=[
            jax.ShapeDtypeStruct((NP, HID), jnp.float32),
            jax.ShapeDtypeStruct((NP, HID), jnp.float32),
        ],
    )(part, nd, w, b, w1a, w1b, b1)


_TB = 3200


def _score_body(z_ref, w_ref, b_ref, o_ref):
    o_ref[...] = jnp.dot(z_ref[...], w_ref[...],
                         preferred_element_type=jnp.float32) + b_ref[...]


def _score_tc(z, w2, b2):
    grid = (TP // _TB,)
    return pl.pallas_call(
        _score_body,
        grid=grid,
        in_specs=[
            pl.BlockSpec((_TB, HID), lambda i: (i, 0)),
            pl.BlockSpec((HID, OUT), lambda i: (0, 0)),
            pl.BlockSpec((1, OUT), lambda i: (0, 0)),
        ],
        out_specs=pl.BlockSpec((_TB, OUT), lambda i: (i, 0)),
        out_shape=jax.ShapeDtypeStruct((TP, OUT), jnp.float32),
    )(z, w2, b2)


def kernel(node_feat, edge_index, triplets, emb, conv_W, conv_b,
           mlp_W1, mlp_b1, mlp_W2, mlp_b2):
    prep, segsum, cntsum, mlp_gather = _sc_kernels()

    nf = jnp.pad(node_feat.astype(jnp.int32), (0, NP - N))
    src = jnp.pad(edge_index[0].astype(jnp.int32), (0, EP - E))
    dst = jnp.pad(edge_index[1].astype(jnp.int32), (0, EP - E),
                  constant_values=N)
    ts = jnp.pad(triplets[:, 0].astype(jnp.int32), (0, TP - T))
    td = jnp.pad(triplets[:, 2].astype(jnp.int32), (0, TP - T))

    h = prep(nf, emb)
    srcc = jnp.pad(edge_index[0].astype(jnp.int32), (0, EP - E),
                   constant_values=N)
    sidx = jnp.concatenate([srcc, dst])
    cnt_tab = jnp.concatenate([
        jnp.tile(jax.nn.one_hot(0, HID, dtype=jnp.float32)[None], (64, 1)),
        jnp.tile(jax.nn.one_hot(1, HID, dtype=jnp.float32)[None], (64, 1)),
        jnp.zeros((64, HID), jnp.float32),
    ])
    cnts = cntsum(sidx, cnt_tab)
    hn0, ns, nd = _norms_tc(cnts, h)
    part0 = segsum(src, dst, hn0)
    hn1 = _upd0(part0, nd, ns, conv_W[0], conv_b[0].reshape(1, HID))
    part1 = segsum(src, dst, hn1)
    p1, p2 = _upd1(part1, nd, conv_W[1], conv_b[1].reshape(1, HID),
                   mlp_W1[:HID], mlp_W1[HID:], mlp_b1.reshape(1, HID))
    z = mlp_gather(ts, td, p1, p2)
    score = _score_tc(z, mlp_W2, mlp_b2.reshape(1, OUT))
    return score[:T]
